# Initial kernel scaffold; baseline (speedup 1.0000x reference)
import functools

import jax
import jax.numpy as jnp
from jax import lax
from jax.experimental import pallas as pl
from jax.experimental.pallas import tpu as pltpu
from jax.experimental.pallas import tpu_sc as plsc

N = 10000
E = 320000
D = 128
CW = 16

NC = 2
NS = 16
NW = NC * NS
EPT = E // NW
CHUNK = 128
NFULL = EPT // CHUNK
TAIL = EPT - NFULL * CHUNK
RPT = N // NS


def _sc_body(x_hbm, src_hbm, dst_hbm, psum_hbm, pcnt_hbm,
             acc, cnt, src_v, dst_v, rows_v, ones_v, zrow, zcnt,
             src_t, dst_t, rows_t, ones_t, sem):
  c = lax.axis_index("c")
  s = lax.axis_index("s")
  wid = s * NC + c

  zeros16 = jnp.zeros((16,), jnp.float32)
  ones16 = jnp.ones((16,), jnp.float32)

  def fill(i, carry):
    for j in range(D // 16):
      zrow[i, pl.ds(j * 16, 16)] = zeros16
    zcnt[i, pl.ds(0, 16)] = zeros16
    ones_v[i, pl.ds(0, 16)] = ones16
    return carry
  lax.fori_loop(0, CHUNK, fill, 0)
  for j in range(TAIL):
    ones_t[j, pl.ds(0, 16)] = ones16

  base = s * RPT
  nz = RPT // CHUNK
  rz = RPT - nz * CHUNK
  for b in range(nz):
    pltpu.sync_copy(zrow, acc.at[pl.ds(base + b * CHUNK, CHUNK)])
    pltpu.sync_copy(zcnt, cnt.at[pl.ds(base + b * CHUNK, CHUNK)])
  pltpu.sync_copy(zrow.at[pl.ds(0, rz)], acc.at[pl.ds(base + nz * CHUNK, rz)])
  pltpu.sync_copy(zcnt.at[pl.ds(0, rz)], cnt.at[pl.ds(base + nz * CHUNK, rz)])

  plsc.subcore_barrier()

  ebase = wid * EPT

  def edge_chunk(i, carry):
    off = ebase + i * CHUNK
    pltpu.sync_copy(src_hbm.at[pl.ds(off, CHUNK)], src_v)
    pltpu.sync_copy(dst_hbm.at[pl.ds(off, CHUNK)], dst_v)
    pltpu.async_copy(x_hbm.at[src_v], rows_v, sem).wait()
    pltpu.sync_copy(rows_v, acc.at[dst_v], add=True)
    pltpu.sync_copy(ones_v, cnt.at[dst_v], add=True)
    return carry
  lax.fori_loop(0, NFULL, edge_chunk, 0)

  toff = ebase + NFULL * CHUNK
  pltpu.sync_copy(src_hbm.at[pl.ds(toff, TAIL)], src_t)
  pltpu.sync_copy(dst_hbm.at[pl.ds(toff, TAIL)], dst_t)
  pltpu.async_copy(x_hbm.at[src_t], rows_t, sem).wait()
  pltpu.sync_copy(rows_t, acc.at[dst_t], add=True)
  pltpu.sync_copy(ones_t, cnt.at[dst_t], add=True)

  plsc.subcore_barrier()

  pltpu.sync_copy(acc.at[pl.ds(base, RPT)], psum_hbm.at[c, pl.ds(base, RPT)])
  pltpu.sync_copy(cnt.at[pl.ds(base, RPT)], pcnt_hbm.at[c, pl.ds(base, RPT)])


@jax.jit
def _sc_aggregate(x, src, dst):
  mesh = plsc.VectorSubcoreMesh(core_axis_name="c", subcore_axis_name="s")
  return pl.kernel(
      _sc_body,
      mesh=mesh,
      out_type=(
          jax.ShapeDtypeStruct((NC, N, D), jnp.float32),
          jax.ShapeDtypeStruct((NC, N, CW), jnp.float32),
      ),
      scratch_types=[
          pltpu.VMEM_SHARED((N, D), jnp.float32),
          pltpu.VMEM_SHARED((N, CW), jnp.float32),
          pltpu.VMEM((CHUNK,), jnp.int32),
          pltpu.VMEM((CHUNK,), jnp.int32),
          pltpu.VMEM((CHUNK, D), jnp.float32),
          pltpu.VMEM((CHUNK, CW), jnp.float32),
          pltpu.VMEM((CHUNK, D), jnp.float32),
          pltpu.VMEM((CHUNK, CW), jnp.float32),
          pltpu.VMEM((TAIL,), jnp.int32),
          pltpu.VMEM((TAIL,), jnp.int32),
          pltpu.VMEM((TAIL, D), jnp.float32),
          pltpu.VMEM((TAIL, CW), jnp.float32),
          pltpu.SemaphoreType.DMA,
      ],
  )(x, src, dst)


def _combine_body(ps_ref, pc_ref, out_ref):
  ssum = ps_ref[0] + ps_ref[1]
  cn = pc_ref[0, :, 0:1] + pc_ref[1, :, 0:1]
  out_ref[...] = ssum / jnp.maximum(cn, 1.0)


@jax.jit
def _tc_combine(psum, pcnt):
  rows = 1250
  grid = N // rows
  return pl.pallas_call(
      _combine_body,
      grid=(grid,),
      in_specs=[
          pl.BlockSpec((NC, rows, D), lambda i: (0, i, 0)),
          pl.BlockSpec((NC, rows, CW), lambda i: (0, i, 0)),
      ],
      out_specs=pl.BlockSpec((rows, D), lambda i: (i, 0)),
      out_shape=jax.ShapeDtypeStruct((N, D), jnp.float32),
  )(psum, pcnt)


def kernel(x, edge_index):
  src = edge_index[0].astype(jnp.int32)
  dst = edge_index[1].astype(jnp.int32)
  psum, pcnt = _sc_aggregate(x, src, dst)
  return _tc_combine(psum, pcnt)


# speedup vs baseline: 6.0274x; 6.0274x over previous
import jax
import jax.numpy as jnp
from jax import lax
from jax.experimental import pallas as pl
from jax.experimental.pallas import tpu as pltpu
from jax.experimental.pallas import tpu_sc as plsc

N = 10000
NP = 10112
E = 320000
D = 128

NC = 2
NS = 16
NW = NC * NS
EPT = E // NW
CHUNK = 128
NFULL = EPT // CHUNK
TAIL = EPT - NFULL * CHUNK
RPT = NP // NS
NZF = RPT // CHUNK
RZ = RPT - NZF * CHUNK


def _sc_body(x_hbm, src_hbm, dst_hbm, psum_hbm, pcnt_hbm,
             acc, src_v, dst_v, rows_v, src_t, dst_t, rows_t, sem):
  c = lax.axis_index("c")
  s = lax.axis_index("s")
  wid = s * NC + c

  zeros16 = jnp.zeros((16,), jnp.float32)
  ones16 = jnp.ones((16,), jnp.float32)

  def fillz(i, carry):
    for j in range(D // 16):
      rows_v[i, pl.ds(j * 16, 16)] = zeros16
    return carry
  lax.fori_loop(0, CHUNK, fillz, 0)

  base = s * RPT
  for b in range(NZF):
    pltpu.sync_copy(rows_v, acc.at[pl.ds(base + b * CHUNK, CHUNK)])
  pltpu.sync_copy(rows_v.at[pl.ds(0, RZ)], acc.at[pl.ds(base + NZF * CHUNK, RZ)])

  plsc.subcore_barrier()

  ebase = wid * EPT

  def edge_chunk(i, carry):
    off = ebase + i * CHUNK
    pltpu.sync_copy(src_hbm.at[pl.ds(off, CHUNK)], src_v)
    pltpu.sync_copy(dst_hbm.at[pl.ds(off, CHUNK)], dst_v)
    pltpu.async_copy(x_hbm.at[src_v], rows_v, sem).wait()
    pltpu.sync_copy(rows_v, acc.at[dst_v], add=True)
    return carry
  lax.fori_loop(0, NFULL, edge_chunk, 0)

  toff = ebase + NFULL * CHUNK
  pltpu.sync_copy(src_hbm.at[pl.ds(toff, TAIL)], src_t)
  pltpu.sync_copy(dst_hbm.at[pl.ds(toff, TAIL)], dst_t)
  pltpu.async_copy(x_hbm.at[src_t], rows_t, sem).wait()
  pltpu.sync_copy(rows_t, acc.at[dst_t], add=True)

  plsc.subcore_barrier()

  hb = c * NP + base
  for b in range(NZF):
    pltpu.sync_copy(acc.at[pl.ds(base + b * CHUNK, CHUNK)], rows_v)
    pltpu.sync_copy(rows_v, psum_hbm.at[pl.ds(hb + b * CHUNK, CHUNK)])
  pltpu.sync_copy(acc.at[pl.ds(base + NZF * CHUNK, RZ)], rows_v.at[pl.ds(0, RZ)])
  pltpu.sync_copy(rows_v.at[pl.ds(0, RZ)], psum_hbm.at[pl.ds(hb + NZF * CHUNK, RZ)])

  lax.fori_loop(0, CHUNK, fillz, 0)
  for b in range(NZF):
    pltpu.sync_copy(rows_v, acc.at[pl.ds(base + b * CHUNK, CHUNK)])
  pltpu.sync_copy(rows_v.at[pl.ds(0, RZ)], acc.at[pl.ds(base + NZF * CHUNK, RZ)])

  plsc.subcore_barrier()

  def fillo(i, carry):
    for j in range(D // 16):
      rows_v[i, pl.ds(j * 16, 16)] = ones16
    return carry
  lax.fori_loop(0, CHUNK, fillo, 0)

  def cnt_chunk(i, carry):
    off = ebase + i * CHUNK
    pltpu.sync_copy(dst_hbm.at[pl.ds(off, CHUNK)], dst_v)
    pltpu.sync_copy(rows_v, acc.at[dst_v], add=True)
    return carry
  lax.fori_loop(0, NFULL, cnt_chunk, 0)

  pltpu.sync_copy(dst_hbm.at[pl.ds(toff, TAIL)], dst_t)
  pltpu.sync_copy(rows_v.at[pl.ds(0, TAIL)], acc.at[dst_t], add=True)

  plsc.subcore_barrier()

  for b in range(NZF):
    pltpu.sync_copy(acc.at[pl.ds(base + b * CHUNK, CHUNK)], rows_v)
    pltpu.sync_copy(rows_v, pcnt_hbm.at[pl.ds(hb + b * CHUNK, CHUNK)])
  pltpu.sync_copy(acc.at[pl.ds(base + NZF * CHUNK, RZ)], rows_v.at[pl.ds(0, RZ)])
  pltpu.sync_copy(rows_v.at[pl.ds(0, RZ)], pcnt_hbm.at[pl.ds(hb + NZF * CHUNK, RZ)])


@jax.jit
def _sc_aggregate(x, src, dst):
  mesh = plsc.VectorSubcoreMesh(core_axis_name="c", subcore_axis_name="s")
  return pl.kernel(
      _sc_body,
      mesh=mesh,
      out_type=(
          jax.ShapeDtypeStruct((NC * NP, D), jnp.float32),
          jax.ShapeDtypeStruct((NC * NP, D), jnp.float32),
      ),
      scratch_types=[
          pltpu.VMEM_SHARED((NP, D), jnp.float32),
          pltpu.VMEM((CHUNK,), jnp.int32),
          pltpu.VMEM((CHUNK,), jnp.int32),
          pltpu.VMEM((CHUNK, D), jnp.float32),
          pltpu.VMEM((TAIL,), jnp.int32),
          pltpu.VMEM((TAIL,), jnp.int32),
          pltpu.VMEM((TAIL, D), jnp.float32),
          pltpu.SemaphoreType.DMA,
      ],
  )(x, src, dst)


def _combine_body(ps_ref, pc_ref, out_ref):
  ssum = ps_ref[0:N, :] + ps_ref[NP:NP + N, :]
  cn = pc_ref[0:N, 0:1] + pc_ref[NP:NP + N, 0:1]
  out_ref[...] = ssum / jnp.maximum(cn, 1.0)


@jax.jit
def _tc_combine(psum, pcnt):
  return pl.pallas_call(
      _combine_body,
      out_shape=jax.ShapeDtypeStruct((N, D), jnp.float32),
  )(psum, pcnt)


def kernel(x, edge_index):
  src = edge_index[0].astype(jnp.int32)
  dst = edge_index[1].astype(jnp.int32)
  psum, pcnt = _sc_aggregate(x, src, dst)
  return _tc_combine(psum, pcnt)


# speedup vs baseline: 6.3868x; 1.0596x over previous
import jax
import jax.numpy as jnp
from jax import lax
from jax.experimental import pallas as pl
from jax.experimental.pallas import tpu as pltpu
from jax.experimental.pallas import tpu_sc as plsc

N = 10000
NP = 10112
E = 320000
D = 128

NC = 2
NS = 16
NW = NC * NS
EPT = E // NW
CHUNK = 128
NFULL = EPT // CHUNK
TAIL = EPT - NFULL * CHUNK
RPT = NP // NS
NZF = RPT // CHUNK
RZ = RPT - NZF * CHUNK


def _sc_body(x_hbm, src_hbm, dst_hbm, psum_hbm, pcnt_hbm,
             acc, src_ring, dst_ring, rows_v, rows_b, src_t, dst_t, rows_t,
             sem, sem2):
  c = lax.axis_index("c")
  s = lax.axis_index("s")
  wid = s * NC + c

  zeros16 = jnp.zeros((16,), jnp.float32)
  ones16 = jnp.ones((16,), jnp.float32)

  def fillz(i, carry):
    for j in range(D // 16):
      rows_v[i, pl.ds(j * 16, 16)] = zeros16
    return carry
  lax.fori_loop(0, CHUNK, fillz, 0)

  base = s * RPT
  for b in range(NZF):
    pltpu.sync_copy(rows_v, acc.at[pl.ds(base + b * CHUNK, CHUNK)])
  pltpu.sync_copy(rows_v.at[pl.ds(0, RZ)], acc.at[pl.ds(base + NZF * CHUNK, RZ)])

  plsc.subcore_barrier()

  ebase = wid * EPT

  def edge_pair(i, carry):
    off0 = ebase + (2 * i) * CHUNK
    off1 = off0 + CHUNK
    pltpu.sync_copy(src_hbm.at[pl.ds(off0, CHUNK)], src_ring.at[0])
    pltpu.sync_copy(dst_hbm.at[pl.ds(off0, CHUNK)], dst_ring.at[0])
    pltpu.sync_copy(src_hbm.at[pl.ds(off1, CHUNK)], src_ring.at[1])
    pltpu.sync_copy(dst_hbm.at[pl.ds(off1, CHUNK)], dst_ring.at[1])
    ga = pltpu.async_copy(x_hbm.at[src_ring.at[0]], rows_v, sem)
    gb = pltpu.async_copy(x_hbm.at[src_ring.at[1]], rows_b, sem2)
    ga.wait()
    pltpu.sync_copy(rows_v, acc.at[dst_ring.at[0]], add=True)
    gb.wait()
    pltpu.sync_copy(rows_b, acc.at[dst_ring.at[1]], add=True)
    return carry
  lax.fori_loop(0, NFULL // 2, edge_pair, 0)

  toff = ebase + NFULL * CHUNK
  pltpu.sync_copy(src_hbm.at[pl.ds(toff, TAIL)], src_t)
  pltpu.sync_copy(dst_hbm.at[pl.ds(toff, TAIL)], dst_t)
  pltpu.async_copy(x_hbm.at[src_t], rows_t, sem).wait()
  pltpu.sync_copy(rows_t, acc.at[dst_t], add=True)

  plsc.subcore_barrier()

  hb = c * NP + base
  for b in range(NZF):
    pltpu.sync_copy(acc.at[pl.ds(base + b * CHUNK, CHUNK)], rows_v)
    pltpu.sync_copy(rows_v, psum_hbm.at[pl.ds(hb + b * CHUNK, CHUNK)])
  pltpu.sync_copy(acc.at[pl.ds(base + NZF * CHUNK, RZ)], rows_v.at[pl.ds(0, RZ)])
  pltpu.sync_copy(rows_v.at[pl.ds(0, RZ)], psum_hbm.at[pl.ds(hb + NZF * CHUNK, RZ)])

  lax.fori_loop(0, CHUNK, fillz, 0)
  for b in range(NZF):
    pltpu.sync_copy(rows_v, acc.at[pl.ds(base + b * CHUNK, CHUNK)])
  pltpu.sync_copy(rows_v.at[pl.ds(0, RZ)], acc.at[pl.ds(base + NZF * CHUNK, RZ)])

  plsc.subcore_barrier()

  def fillo(i, carry):
    for j in range(D // 16):
      rows_v[i, pl.ds(j * 16, 16)] = ones16
      rows_b[i, pl.ds(j * 16, 16)] = ones16
    return carry
  lax.fori_loop(0, CHUNK, fillo, 0)

  def cnt_pair(i, carry):
    off0 = ebase + (2 * i) * CHUNK
    off1 = off0 + CHUNK
    pltpu.sync_copy(dst_hbm.at[pl.ds(off0, CHUNK)], dst_ring.at[0])
    pltpu.sync_copy(dst_hbm.at[pl.ds(off1, CHUNK)], dst_ring.at[1])
    s0 = pltpu.async_copy(rows_v, acc.at[dst_ring.at[0]], sem, add=True)
    s1 = pltpu.async_copy(rows_b, acc.at[dst_ring.at[1]], sem2, add=True)
    s0.wait()
    s1.wait()
    return carry
  lax.fori_loop(0, NFULL // 2, cnt_pair, 0)

  pltpu.sync_copy(dst_hbm.at[pl.ds(toff, TAIL)], dst_t)
  pltpu.sync_copy(rows_v.at[pl.ds(0, TAIL)], acc.at[dst_t], add=True)

  plsc.subcore_barrier()

  for b in range(NZF):
    pltpu.sync_copy(acc.at[pl.ds(base + b * CHUNK, CHUNK)], rows_v)
    pltpu.sync_copy(rows_v, pcnt_hbm.at[pl.ds(hb + b * CHUNK, CHUNK)])
  pltpu.sync_copy(acc.at[pl.ds(base + NZF * CHUNK, RZ)], rows_v.at[pl.ds(0, RZ)])
  pltpu.sync_copy(rows_v.at[pl.ds(0, RZ)], pcnt_hbm.at[pl.ds(hb + NZF * CHUNK, RZ)])


@jax.jit
def _sc_aggregate(x, src, dst):
  mesh = plsc.VectorSubcoreMesh(core_axis_name="c", subcore_axis_name="s")
  return pl.kernel(
      _sc_body,
      mesh=mesh,
      out_type=(
          jax.ShapeDtypeStruct((NC * NP, D), jnp.float32),
          jax.ShapeDtypeStruct((NC * NP, D), jnp.float32),
      ),
      scratch_types=[
          pltpu.VMEM_SHARED((NP, D), jnp.float32),
          pltpu.VMEM((2, CHUNK), jnp.int32),
          pltpu.VMEM((2, CHUNK), jnp.int32),
          pltpu.VMEM((CHUNK, D), jnp.float32),
          pltpu.VMEM((CHUNK, D), jnp.float32),
          pltpu.VMEM((TAIL,), jnp.int32),
          pltpu.VMEM((TAIL,), jnp.int32),
          pltpu.VMEM((TAIL, D), jnp.float32),
          pltpu.SemaphoreType.DMA,
          pltpu.SemaphoreType.DMA,
      ],
  )(x, src, dst)


def _combine_body(ps_ref, pc_ref, out_ref):
  ssum = ps_ref[0:N, :] + ps_ref[NP:NP + N, :]
  cn = pc_ref[0:N, 0:1] + pc_ref[NP:NP + N, 0:1]
  out_ref[...] = ssum / jnp.maximum(cn, 1.0)


@jax.jit
def _tc_combine(psum, pcnt):
  return pl.pallas_call(
      _combine_body,
      out_shape=jax.ShapeDtypeStruct((N, D), jnp.float32),
  )(psum, pcnt)


def kernel(x, edge_index):
  src = edge_index[0].astype(jnp.int32)
  dst = edge_index[1].astype(jnp.int32)
  psum, pcnt = _sc_aggregate(x, src, dst)
  return _tc_combine(psum, pcnt)
